# baseline (speedup 1.0000x reference)
import functools

import jax
import jax.numpy as jnp
from jax.experimental import pallas as pl
from jax.experimental.pallas import tpu as pltpu


def _leaky(v, slope):
    return jnp.where(v >= 0, v, v * slope)


def _proj_kernel(x_ref, w_ref, a_ref, waug_ref, e_ref, *, heads, o, ow):
    wh = jnp.dot(x_ref[...], w_ref[...], preferred_element_type=jnp.float32)
    bn = wh.shape[0]
    parts = []
    for h in range(heads):
        parts.append(wh[:, h * o:(h + 1) * o])
        parts.append(jnp.ones((bn, 1), jnp.float32))
        if ow > o + 1:
            parts.append(jnp.zeros((bn, ow - o - 1), jnp.float32))
    waug_ref[...] = jnp.concatenate(parts, axis=1).astype(jnp.bfloat16)
    s = jnp.dot(wh, a_ref[...], preferred_element_type=jnp.float32)
    e_ref[...] = jnp.exp(jnp.concatenate([s, 0.2 * s], axis=1))


def _project(x, wf, af, bn, ow):
    n, f = x.shape
    ho = wf.shape[1]
    a4 = af.shape[1]
    heads = a4 // 2
    o = ho // heads
    return pl.pallas_call(
        functools.partial(_proj_kernel, heads=heads, o=o, ow=ow),
        grid=(n // bn,),
        in_specs=[
            pl.BlockSpec((bn, f), lambda b: (b, 0)),
            pl.BlockSpec((f, ho), lambda b: (0, 0)),
            pl.BlockSpec((ho, a4), lambda b: (0, 0)),
        ],
        out_specs=[
            pl.BlockSpec((bn, heads * ow), lambda b: (b, 0)),
            pl.BlockSpec((bn, 2 * a4), lambda b: (b, 0)),
        ],
        out_shape=[
            jax.ShapeDtypeStruct((n, heads * ow), jnp.bfloat16),
            jax.ShapeDtypeStruct((n, 2 * a4), jnp.float32),
        ],
    )(x, wf, af)


def _flash_kernel(ssrc_ref, sdt_ref, adj_ref, wh_ref, b_ref, out_ref,
                  acc, *, heads, o, ow, n):
    j = pl.program_id(1)
    bj = adj_ref.shape[1]

    @pl.when(j == 0)
    def _():
        acc[...] = jnp.zeros_like(acc)

    col = jax.lax.broadcasted_iota(jnp.int32, (1, bj), 1) + j * bj
    adjm = jnp.where(col < n, adj_ref[...], 0.0)
    for h in range(heads):
        u = ssrc_ref[:, h:h + 1] * sdt_ref[h:h + 1, :]
        v = ssrc_ref[:, heads + h:heads + h + 1] * sdt_ref[heads + h:heads + h + 1, :]
        p = (adjm * jnp.maximum(u, v)).astype(jnp.bfloat16)
        acc[:, h * ow:(h + 1) * ow] += jnp.dot(
            p, wh_ref[pl.ds(j * bj, bj), h * ow:(h + 1) * ow],
            preferred_element_type=jnp.float32)

    @pl.when(j == pl.num_programs(1) - 1)
    def _():
        parts = [acc[:, h * ow:h * ow + o] / acc[:, h * ow + o:h * ow + o + 1]
                 for h in range(heads)]
        out = jnp.concatenate(parts, axis=1) + b_ref[...]
        out_ref[...] = _leaky(out, 0.01)


def _gat_layer(waug, e, adj, bias, heads, o, ow, bi, bj):
    n = adj.shape[0]
    ho = heads * o
    nj = -(-n // bj)
    npad = nj * bj - n
    h2 = 2 * heads
    src_e = jnp.concatenate([e[:, :heads], e[:, h2:h2 + heads]], axis=1)
    dst_et = jnp.concatenate([e[:, heads:h2], e[:, h2 + heads:]], axis=1).T
    dst_et = jnp.pad(dst_et, ((0, 0), (0, npad)))
    waug = jnp.pad(waug, ((0, npad), (0, 0)))
    return pl.pallas_call(
        functools.partial(_flash_kernel, heads=heads, o=o, ow=ow, n=n),
        grid=(n // bi, nj),
        in_specs=[
            pl.BlockSpec((bi, h2), lambda i, j: (i, 0)),
            pl.BlockSpec((h2, bj), lambda i, j: (0, j)),
            pl.BlockSpec((bi, bj), lambda i, j: (i, j)),
            pl.BlockSpec((n + npad, heads * ow), lambda i, j: (0, 0)),
            pl.BlockSpec((1, ho), lambda i, j: (0, 0)),
        ],
        out_specs=pl.BlockSpec((bi, ho), lambda i, j: (i, 0)),
        out_shape=jax.ShapeDtypeStruct((n, ho), jnp.float32),
        scratch_shapes=[
            pltpu.VMEM((bi, heads * ow), jnp.float32),
        ],
        compiler_params=pltpu.CompilerParams(
            dimension_semantics=("parallel", "arbitrary")),
    )(src_e, dst_et, adj, waug, bias)


def _pack_w(w):
    h, f, o = w.shape
    return jnp.transpose(w, (1, 0, 2)).reshape(f, h * o)


def _pack_a(a_src, a_dst):
    h, o = a_src.shape
    z = jnp.zeros((h * o, 2 * h), jnp.float32)
    for i in range(h):
        z = z.at[i * o:(i + 1) * o, i].set(a_src[i])
        z = z.at[i * o:(i + 1) * o, h + i].set(a_dst[i])
    return z


def _block_sizes(n):
    if n % 2000 == 0:
        return 2000, 1000, 2048
    return n, n, n


def kernel(x, adj, W1, a_src1, a_dst1, b1, W2, a_src2, a_dst2, b2):
    heads = W1.shape[0]
    bn, bi, bj = _block_sizes(adj.shape[0])
    o1, o2 = W1.shape[2], W2.shape[2]
    ow1 = 128 * (-(-(o1 + 1) // 128))
    ow2 = 128 * (-(-(o2 + 1) // 128))

    waug1, e1 = _project(x, _pack_w(W1), _pack_a(a_src1, a_dst1), bn, ow1)
    h1 = _gat_layer(waug1, e1, adj, b1.reshape(1, -1), heads, o1, ow1, bi, bj)

    waug2, e2 = _project(h1, _pack_w(W2), _pack_a(a_src2, a_dst2), bn, ow2)
    return _gat_layer(waug2, e2, adj, b2.reshape(1, -1), heads, o2, ow2, bi, bj)
